# baseline (speedup 1.0000x reference)
import functools

import jax
import jax.numpy as jnp
from jax import lax
from jax.experimental import pallas as pl
from jax.experimental.pallas import tpu as pltpu
from jax.experimental.pallas import tpu_sc as plsc

_B = 4096
_L = 20
_D = 16
_BL = _B * _L
_NW = 32
_CHUNK = 128
_ROWS_W = _BL // _NW
_CH_W = _ROWS_W // _CHUNK
_PASSES = 2
_CH_P = _CH_W // _PASSES
_M = _CH_P * _CHUNK


@functools.partial(
    pl.kernel,
    mesh=plsc.VectorSubcoreMesh(core_axis_name="c", subcore_axis_name="s"),
    out_type=jax.ShapeDtypeStruct((_BL, 3 * _D), jnp.float32),
    compiler_params=pltpu.CompilerParams(use_tc_tiling_on_sc=False),
    scratch_types=[
        pltpu.VMEM((_M,), jnp.int32),
        pltpu.VMEM((_M,), jnp.int32),
        pltpu.VMEM((_M,), jnp.int32),
        pltpu.VMEM((_M, _D), jnp.float32),
        pltpu.VMEM((_M, _D), jnp.float32),
        pltpu.VMEM((_M, _D), jnp.float32),
        pltpu.SemaphoreType.DMA,
    ],
)
def _gather_concat(iid_hbm, a0_hbm, a1_hbm, item_t, attr_t, out_hbm,
                   idx_i, idx_a0, idx_a1, item_v, a0_v, a1_v, sem):
    wid = lax.axis_index("s") * 2 + lax.axis_index("c")
    for p in range(_PASSES):
        row0 = (wid * _PASSES + p) * _M
        pltpu.sync_copy(iid_hbm.at[pl.ds(row0, _M)], idx_i)
        pltpu.sync_copy(a0_hbm.at[pl.ds(row0, _M)], idx_a0)
        pltpu.sync_copy(a1_hbm.at[pl.ds(row0, _M)], idx_a1)
        c1 = pltpu.async_copy(item_t.at[idx_i], item_v, sem)
        c2 = pltpu.async_copy(attr_t.at[idx_a0], a0_v, sem)
        c3 = pltpu.async_copy(attr_t.at[idx_a1], a1_v, sem)
        c1.wait()
        c2.wait()
        c3.wait()
        pltpu.sync_copy(item_v, out_hbm.at[pl.ds(row0, _M), pl.ds(0, _D)])
        pltpu.sync_copy(a0_v, out_hbm.at[pl.ds(row0, _M), pl.ds(_D, _D)])
        pltpu.sync_copy(a1_v, out_hbm.at[pl.ds(row0, _M), pl.ds(2 * _D, _D)])


def kernel(hist_iid_seq, hist_aid_seq, hist_rate_seq, hist_seq_len,
           iid_list, aid_list, lb_list,
           item_table, attr_table, rating_table):
    iid = iid_list.reshape(_BL).astype(jnp.int32)
    a0 = aid_list[:, :, 0].reshape(_BL).astype(jnp.int32)
    a1 = aid_list[:, :, 1].reshape(_BL).astype(jnp.int32)
    out = _gather_concat(iid, a0, a1, item_table, attr_table)
    return out.reshape(_B, _L, 3 * _D), lb_list.reshape(_B, _L)
